# baseline (speedup 1.0000x reference)
import functools

import jax
import jax.numpy as jnp
from jax import lax
from jax.experimental import pallas as pl
from jax.experimental.pallas import tpu as pltpu
from jax.experimental.pallas import tpu_sc as plsc

D_MODEL = 128
FRAG_DIM = 64
ATOM_DIM = 64
LANES = 16
NB_SPMEM = 5


@functools.lru_cache(maxsize=None)
def _build(B, F, A):
    R = F * A
    NC, NS = 2, 16
    NW = NC * NS
    BPW = B // NW
    NCH = 10
    CF = F // NCH
    CR = CF * A
    SR = R // NS

    mesh = plsc.VectorSubcoreMesh(core_axis_name="c", subcore_axis_name="s")

    @functools.partial(
        pl.kernel,
        out_type=[
            jax.ShapeDtypeStruct((B, F, D_MODEL), jnp.float32),
            jax.ShapeDtypeStruct((B, R, D_MODEL), jnp.float32),
        ],
        mesh=mesh,
        scratch_types=[
            pltpu.VMEM((F, FRAG_DIM), jnp.float32),
            pltpu.VMEM((A, ATOM_DIM), jnp.float32),
            pltpu.VMEM((F, D_MODEL), jnp.float32),
            pltpu.VMEM((CR, D_MODEL), jnp.float32),
            pltpu.VMEM((CR, D_MODEL), jnp.float32),
            pltpu.VMEM_SHARED((R, D_MODEL), jnp.float32),
            pltpu.SemaphoreType.DMA,
            pltpu.SemaphoreType.DMA,
            pltpu.SemaphoreType.DMA,
            pltpu.SemaphoreType.DMA,
        ],
    )
    def sc_kernel(pe_frag_hbm, pe_atom_hbm, frag_out_hbm, atom_out_hbm,
                  pf_v, pa_v, fo_v, ch0_v, ch1_v, pat_sh,
                  sem0, sem1, fsem, ssem):
        cid = lax.axis_index("c")
        sid = lax.axis_index("s")
        wid = sid * NC + cid
        b0 = wid * BPW

        pltpu.sync_copy(pe_frag_hbm, pf_v)
        pltpu.sync_copy(pe_atom_hbm, pa_v)

        zero = jnp.zeros((LANES,), jnp.float32)
        bufs = (ch0_v, ch1_v)
        sems = (sem0, sem1)

        r0 = sid * SR

        def pat_row(r, carry):
            f = lax.div(r0 + r, A)
            a = lax.rem(r0 + r, A)
            for j in range(FRAG_DIM // LANES):
                ch0_v[r, pl.ds(j * LANES, LANES)] = pf_v[f, pl.ds(j * LANES, LANES)]
            for j in range(ATOM_DIM // LANES):
                ch0_v[r, pl.ds(FRAG_DIM + j * LANES, LANES)] = pa_v[a, pl.ds(j * LANES, LANES)]
            return carry

        lax.fori_loop(0, SR, pat_row, 0)
        pltpu.sync_copy(ch0_v.at[pl.ds(0, SR), :], pat_sh.at[pl.ds(r0, SR), :])
        plsc.subcore_barrier()

        sp_descs = [
            pltpu.async_copy(pat_sh, atom_out_hbm.at[b0 + i], ssem)
            for i in range(NB_SPMEM)
        ]

        def fo_row(r, carry):
            for j in range(FRAG_DIM // LANES):
                fo_v[r, pl.ds(j * LANES, LANES)] = pf_v[r, pl.ds(j * LANES, LANES)]
            for j in range(ATOM_DIM // LANES):
                fo_v[r, pl.ds(FRAG_DIM + j * LANES, LANES)] = zero
            return carry

        lax.fori_loop(0, F, fo_row, 0)

        fo_descs = [
            pltpu.async_copy(fo_v, frag_out_hbm.at[b0 + i], fsem)
            for i in range(BPW)
        ]

        def atom_row(buf):
            def body(r, carry):
                a = lax.rem(r, A)
                for j in range(ATOM_DIM // LANES):
                    buf[r, pl.ds(FRAG_DIM + j * LANES, LANES)] = pa_v[a, pl.ds(j * LANES, LANES)]
                return carry

            lax.fori_loop(0, CR, body, 0)

        atom_row(ch0_v)
        atom_row(ch1_v)

        pending = [[], []]
        for ci in range(NCH):
            k = ci % 2
            buf = bufs[k]
            for d in pending[k]:
                d.wait()
            pending[k] = []

            f0 = ci * CF

            def frag_row(r, carry, buf=buf, f0=f0):
                f = f0 + lax.div(r, A)
                for j in range(FRAG_DIM // LANES):
                    buf[r, pl.ds(j * LANES, LANES)] = pf_v[f, pl.ds(j * LANES, LANES)]
                return carry

            lax.fori_loop(0, CR, frag_row, 0)

            pending[k] = [
                pltpu.async_copy(
                    buf, atom_out_hbm.at[b0 + i, pl.ds(ci * CR, CR), :], sems[k])
                for i in range(NB_SPMEM, BPW)
            ]

        for k in (0, 1):
            for d in pending[k]:
                d.wait()
        for d in fo_descs:
            d.wait()
        for d in sp_descs:
            d.wait()

    return sc_kernel


def kernel(fragment_features, atom_features, pe_frag, pe_atom):
    B, F = fragment_features.shape[:2]
    A = atom_features.shape[1] // F
    fn = _build(B, F, A)
    frag_out, atom_full = fn(pe_frag, pe_atom)
    return (frag_out, atom_full)


# speedup vs baseline: 1.0778x; 1.0778x over previous
import functools

import jax
import jax.numpy as jnp
from jax import lax
from jax.experimental import pallas as pl
from jax.experimental.pallas import tpu as pltpu
from jax.experimental.pallas import tpu_sc as plsc

D_MODEL = 128
FRAG_DIM = 64
ATOM_DIM = 64
LANES = 16
NB_SPMEM = 4


@functools.lru_cache(maxsize=None)
def _build(B, F, A):
    R = F * A
    NC, NS = 2, 16
    NW = NC * NS
    BPW = B // NW
    NCH = 10
    CF = F // NCH
    CR = CF * A
    SR = R // NS

    mesh = plsc.VectorSubcoreMesh(core_axis_name="c", subcore_axis_name="s")

    @functools.partial(
        pl.kernel,
        out_type=[
            jax.ShapeDtypeStruct((B, F, D_MODEL), jnp.float32),
            jax.ShapeDtypeStruct((B, R, D_MODEL), jnp.float32),
        ],
        mesh=mesh,
        scratch_types=[
            pltpu.VMEM((F, FRAG_DIM), jnp.float32),
            pltpu.VMEM((A, ATOM_DIM), jnp.float32),
            pltpu.VMEM((F, D_MODEL), jnp.float32),
            pltpu.VMEM((CR, D_MODEL), jnp.float32),
            pltpu.VMEM((CR, D_MODEL), jnp.float32),
            pltpu.VMEM_SHARED((R, D_MODEL), jnp.float32),
            pltpu.SemaphoreType.DMA,
            pltpu.SemaphoreType.DMA,
            pltpu.SemaphoreType.DMA,
            pltpu.SemaphoreType.DMA,
        ],
    )
    def sc_kernel(pe_frag_hbm, pe_atom_hbm, frag_out_hbm, atom_out_hbm,
                  pf_v, pa_v, fo_v, ch0_v, ch1_v, pat_sh,
                  sem0, sem1, fsem, ssem):
        cid = lax.axis_index("c")
        sid = lax.axis_index("s")
        wid = sid * NC + cid
        b0 = wid * BPW

        pltpu.sync_copy(pe_frag_hbm, pf_v)
        pltpu.sync_copy(pe_atom_hbm, pa_v)

        zero = jnp.zeros((LANES,), jnp.float32)
        bufs = (ch0_v, ch1_v)
        sems = (sem0, sem1)

        def atom_half(buf):
            def body(r, carry):
                a = lax.rem(r, A)
                for j in range(ATOM_DIM // LANES):
                    buf[r, pl.ds(FRAG_DIM + j * LANES, LANES)] = pa_v[a, pl.ds(j * LANES, LANES)]
                return carry

            lax.fori_loop(0, CR, body, 0)

        def frag_half(buf, f0):
            def grp(g, carry):
                vs = [pf_v[f0 + g, pl.ds(j * LANES, LANES)]
                      for j in range(FRAG_DIM // LANES)]

                def inner(a, c2):
                    r = g * A + a
                    for j in range(FRAG_DIM // LANES):
                        buf[r, pl.ds(j * LANES, LANES)] = vs[j]
                    return c2

                lax.fori_loop(0, A, inner, 0)
                return carry

            lax.fori_loop(0, CF, grp, 0)

        def fire_chunk(buf, ci, sem):
            return [
                pltpu.async_copy(
                    buf, atom_out_hbm.at[b0 + i, pl.ds(ci * CR, CR), :], sem)
                for i in range(NB_SPMEM, BPW)
            ]

        atom_half(ch0_v)
        frag_half(ch0_v, 0)
        pending = [fire_chunk(ch0_v, 0, sem0), []]

        r0 = sid * SR

        def pat_row(r, carry):
            f = lax.div(r0 + r, A)
            a = lax.rem(r0 + r, A)
            for j in range(FRAG_DIM // LANES):
                ch1_v[r, pl.ds(j * LANES, LANES)] = pf_v[f, pl.ds(j * LANES, LANES)]
            for j in range(ATOM_DIM // LANES):
                ch1_v[r, pl.ds(FRAG_DIM + j * LANES, LANES)] = pa_v[a, pl.ds(j * LANES, LANES)]
            return carry

        lax.fori_loop(0, SR, pat_row, 0)
        pltpu.sync_copy(ch1_v.at[pl.ds(0, SR), :], pat_sh.at[pl.ds(r0, SR), :])
        plsc.subcore_barrier()

        sp_descs = [
            pltpu.async_copy(pat_sh, atom_out_hbm.at[b0 + i], ssem)
            for i in range(NB_SPMEM)
        ]

        atom_half(ch1_v)
        for ci in range(1, NCH):
            k = ci % 2
            buf = bufs[k]
            for d in pending[k]:
                d.wait()
            frag_half(buf, ci * CF)
            pending[k] = fire_chunk(buf, ci, sems[k])

        def fo_row(r, carry):
            for j in range(FRAG_DIM // LANES):
                fo_v[r, pl.ds(j * LANES, LANES)] = pf_v[r, pl.ds(j * LANES, LANES)]
            for j in range(ATOM_DIM // LANES):
                fo_v[r, pl.ds(FRAG_DIM + j * LANES, LANES)] = zero
            return carry

        lax.fori_loop(0, F, fo_row, 0)

        fo_descs = [
            pltpu.async_copy(fo_v, frag_out_hbm.at[b0 + i], fsem)
            for i in range(BPW)
        ]

        for k in (0, 1):
            for d in pending[k]:
                d.wait()
        for d in fo_descs:
            d.wait()
        for d in sp_descs:
            d.wait()

    return sc_kernel


def kernel(fragment_features, atom_features, pe_frag, pe_atom):
    B, F = fragment_features.shape[:2]
    A = atom_features.shape[1] // F
    fn = _build(B, F, A)
    frag_out, atom_full = fn(pe_frag, pe_atom)
    return (frag_out, atom_full)


# speedup vs baseline: 1.1008x; 1.0213x over previous
import functools

import jax
import jax.numpy as jnp
from jax import lax
from jax.experimental import pallas as pl
from jax.experimental.pallas import tpu as pltpu
from jax.experimental.pallas import tpu_sc as plsc

D_MODEL = 128
FRAG_DIM = 64
ATOM_DIM = 64
LANES = 16
NB_SPMEM = 4


@functools.lru_cache(maxsize=None)
def _build(B, F, A):
    R = F * A
    NC, NS = 2, 16
    NW = NC * NS
    BPW = B // NW
    NCH = 10
    CF = F // NCH
    CR = CF * A
    SR = R // NS

    mesh = plsc.VectorSubcoreMesh(core_axis_name="c", subcore_axis_name="s")

    @functools.partial(
        pl.kernel,
        out_type=[
            jax.ShapeDtypeStruct((B, F, D_MODEL), jnp.float32),
            jax.ShapeDtypeStruct((B, R, D_MODEL), jnp.float32),
        ],
        mesh=mesh,
        scratch_types=[
            pltpu.VMEM((F, FRAG_DIM), jnp.float32),
            pltpu.VMEM((A, ATOM_DIM), jnp.float32),
            pltpu.VMEM((F, D_MODEL), jnp.float32),
            pltpu.VMEM((CR, D_MODEL), jnp.float32),
            pltpu.VMEM((CR, D_MODEL), jnp.float32),
            pltpu.VMEM_SHARED((R, D_MODEL), jnp.float32),
            pltpu.SemaphoreType.DMA,
            pltpu.SemaphoreType.DMA,
            pltpu.SemaphoreType.DMA,
            pltpu.SemaphoreType.DMA,
        ],
    )
    def sc_kernel(pe_frag_hbm, pe_atom_hbm, frag_out_hbm, atom_out_hbm,
                  pf_v, pa_v, fo_v, ch0_v, ch1_v, pat_sh,
                  sem0, sem1, fsem, ssem):
        cid = lax.axis_index("c")
        sid = lax.axis_index("s")
        wid = sid * NC + cid
        b0 = wid * BPW

        pltpu.sync_copy(pe_frag_hbm, pf_v)
        pltpu.sync_copy(pe_atom_hbm, pa_v)

        zero = jnp.zeros((LANES,), jnp.float32)
        bufs = (ch0_v, ch1_v)
        sems = (sem0, sem1)

        def atom_half(buf):
            def body(r, carry):
                a = lax.rem(r, A)
                for j in range(ATOM_DIM // LANES):
                    buf[r, pl.ds(FRAG_DIM + j * LANES, LANES)] = pa_v[a, pl.ds(j * LANES, LANES)]
                return carry

            lax.fori_loop(0, CR, body, 0)

        def frag_half(buf, f0):
            def grp(g, carry):
                vs = [pf_v[f0 + g, pl.ds(j * LANES, LANES)]
                      for j in range(FRAG_DIM // LANES)]

                def inner(a, c2):
                    r = g * A + a
                    for j in range(FRAG_DIM // LANES):
                        buf[r, pl.ds(j * LANES, LANES)] = vs[j]
                    return c2

                lax.fori_loop(0, A, inner, 0)
                return carry

            lax.fori_loop(0, CF, grp, 0)

        def fire_chunk(buf, ci, sem):
            return [
                pltpu.async_copy(
                    buf, atom_out_hbm.at[b0 + i, pl.ds(ci * CR, CR), :], sem)
                for i in range(NB_SPMEM, BPW)
            ]

        r0 = sid * SR

        def pat_row(r, carry):
            f = lax.div(r0 + r, A)
            a = lax.rem(r0 + r, A)
            for j in range(FRAG_DIM // LANES):
                ch1_v[r, pl.ds(j * LANES, LANES)] = pf_v[f, pl.ds(j * LANES, LANES)]
            for j in range(ATOM_DIM // LANES):
                ch1_v[r, pl.ds(FRAG_DIM + j * LANES, LANES)] = pa_v[a, pl.ds(j * LANES, LANES)]
            return carry

        lax.fori_loop(0, SR, pat_row, 0)
        pltpu.sync_copy(ch1_v.at[pl.ds(0, SR), :], pat_sh.at[pl.ds(r0, SR), :])
        plsc.subcore_barrier()

        sp_descs = [
            pltpu.async_copy(pat_sh, atom_out_hbm.at[b0 + i], ssem)
            for i in range(NB_SPMEM)
        ]

        atom_half(ch0_v)
        frag_half(ch0_v, 0)
        pending = [fire_chunk(ch0_v, 0, sem0), []]
        atom_half(ch1_v)
        for ci in range(1, NCH):
            k = ci % 2
            buf = bufs[k]
            for d in pending[k]:
                d.wait()
            frag_half(buf, ci * CF)
            pending[k] = fire_chunk(buf, ci, sems[k])

        def fo_row(r, carry):
            for j in range(FRAG_DIM // LANES):
                fo_v[r, pl.ds(j * LANES, LANES)] = pf_v[r, pl.ds(j * LANES, LANES)]
            for j in range(ATOM_DIM // LANES):
                fo_v[r, pl.ds(FRAG_DIM + j * LANES, LANES)] = zero
            return carry

        lax.fori_loop(0, F, fo_row, 0)

        fo_descs = [
            pltpu.async_copy(fo_v, frag_out_hbm.at[b0 + i], fsem)
            for i in range(BPW)
        ]

        for k in (0, 1):
            for d in pending[k]:
                d.wait()
        for d in fo_descs:
            d.wait()
        for d in sp_descs:
            d.wait()

    return sc_kernel


def kernel(fragment_features, atom_features, pe_frag, pe_atom):
    B, F = fragment_features.shape[:2]
    A = atom_features.shape[1] // F
    fn = _build(B, F, A)
    frag_out, atom_full = fn(pe_frag, pe_atom)
    return (frag_out, atom_full)


# speedup vs baseline: 1.1078x; 1.0064x over previous
import functools

import jax
import jax.numpy as jnp
from jax import lax
from jax.experimental import pallas as pl
from jax.experimental.pallas import tpu as pltpu
from jax.experimental.pallas import tpu_sc as plsc

D_MODEL = 128
FRAG_DIM = 64
ATOM_DIM = 64
LANES = 16
NB_SPMEM = 4


@functools.lru_cache(maxsize=None)
def _build(B, F, A):
    R = F * A
    NC, NS = 2, 16
    NW = NC * NS
    BPW = B // NW
    NCH = 10
    CF = F // NCH
    CR = CF * A
    SR = R // NS

    mesh = plsc.VectorSubcoreMesh(core_axis_name="c", subcore_axis_name="s")

    @functools.partial(
        pl.kernel,
        out_type=[
            jax.ShapeDtypeStruct((B, R, D_MODEL), jnp.float32),
        ],
        mesh=mesh,
        scratch_types=[
            pltpu.VMEM((F, FRAG_DIM), jnp.float32),
            pltpu.VMEM((A, ATOM_DIM), jnp.float32),
            pltpu.VMEM((CR, D_MODEL), jnp.float32),
            pltpu.VMEM((CR, D_MODEL), jnp.float32),
            pltpu.VMEM_SHARED((R, D_MODEL), jnp.float32),
            pltpu.SemaphoreType.DMA,
            pltpu.SemaphoreType.DMA,
            pltpu.SemaphoreType.DMA,
        ],
    )
    def sc_kernel(pe_frag_hbm, pe_atom_hbm, atom_out_hbm,
                  pf_v, pa_v, ch0_v, ch1_v, pat_sh,
                  sem0, sem1, ssem):
        cid = lax.axis_index("c")
        sid = lax.axis_index("s")
        wid = sid * NC + cid
        b0 = wid * BPW

        pltpu.sync_copy(pe_frag_hbm, pf_v)
        pltpu.sync_copy(pe_atom_hbm, pa_v)

        bufs = (ch0_v, ch1_v)
        sems = (sem0, sem1)

        def atom_half(buf):
            def body(r, carry):
                a = lax.rem(r, A)
                for j in range(ATOM_DIM // LANES):
                    buf[r, pl.ds(FRAG_DIM + j * LANES, LANES)] = pa_v[a, pl.ds(j * LANES, LANES)]
                return carry

            lax.fori_loop(0, CR, body, 0)

        def frag_half(buf, f0):
            def grp(g, carry):
                vs = [pf_v[f0 + g, pl.ds(j * LANES, LANES)]
                      for j in range(FRAG_DIM // LANES)]

                def inner(a, c2):
                    r = g * A + a
                    for j in range(FRAG_DIM // LANES):
                        buf[r, pl.ds(j * LANES, LANES)] = vs[j]
                    return c2

                lax.fori_loop(0, A, inner, 0)
                return carry

            lax.fori_loop(0, CF, grp, 0)

        def fire_chunk(buf, ci, sem):
            return [
                pltpu.async_copy(
                    buf, atom_out_hbm.at[b0 + i, pl.ds(ci * CR, CR), :], sem)
                for i in range(NB_SPMEM, BPW)
            ]

        r0 = sid * SR

        def pat_row(r, carry):
            f = lax.div(r0 + r, A)
            a = lax.rem(r0 + r, A)
            for j in range(FRAG_DIM // LANES):
                ch1_v[r, pl.ds(j * LANES, LANES)] = pf_v[f, pl.ds(j * LANES, LANES)]
            for j in range(ATOM_DIM // LANES):
                ch1_v[r, pl.ds(FRAG_DIM + j * LANES, LANES)] = pa_v[a, pl.ds(j * LANES, LANES)]
            return carry

        lax.fori_loop(0, SR, pat_row, 0)
        pltpu.sync_copy(ch1_v.at[pl.ds(0, SR), :], pat_sh.at[pl.ds(r0, SR), :])
        plsc.subcore_barrier()

        sp_descs = [
            pltpu.async_copy(pat_sh, atom_out_hbm.at[b0 + i], ssem)
            for i in range(NB_SPMEM)
        ]

        atom_half(ch0_v)
        frag_half(ch0_v, 0)
        pending = [fire_chunk(ch0_v, 0, sem0), []]
        atom_half(ch1_v)
        for ci in range(1, NCH):
            k = ci % 2
            buf = bufs[k]
            for d in pending[k]:
                d.wait()
            frag_half(buf, ci * CF)
            pending[k] = fire_chunk(buf, ci, sems[k])

        for k in (0, 1):
            for d in pending[k]:
                d.wait()
        for d in sp_descs:
            d.wait()

    def fo_body(pf_ref, out_ref):
        fp = pf_ref[...]
        pad = jnp.zeros((F, ATOM_DIM), jnp.float32)
        out_ref[0] = jnp.concatenate([fp, pad], axis=-1)

    fo_call = pl.pallas_call(
        fo_body,
        grid=(B,),
        in_specs=[pl.BlockSpec((F, FRAG_DIM), lambda b: (0, 0))],
        out_specs=pl.BlockSpec((1, F, D_MODEL), lambda b: (b, 0, 0)),
        out_shape=jax.ShapeDtypeStruct((B, F, D_MODEL), jnp.float32),
    )

    def run(pe_frag, pe_atom):
        res = sc_kernel(pe_frag, pe_atom)
        atom_full = res[0] if isinstance(res, (list, tuple)) else res
        frag_out = fo_call(pe_frag)
        return frag_out, atom_full

    return run


def kernel(fragment_features, atom_features, pe_frag, pe_atom):
    B, F = fragment_features.shape[:2]
    A = atom_features.shape[1] // F
    fn = _build(B, F, A)
    frag_out, atom_full = fn(pe_frag, pe_atom)
    return (frag_out, atom_full)
